# baseline (speedup 1.0000x reference)
import functools

import jax
import jax.numpy as jnp
from jax import lax
from jax.experimental import pallas as pl
from jax.experimental.pallas import tpu as pltpu
from jax.experimental.pallas import tpu_sc as plsc

_NC, _NS, _L = 2, 16, 16
_NW = _NC * _NS
_N = 16384 * 128
_PER_W = _N // _NW
_CHUNK = 8192
_NCHUNK = _PER_W // _CHUNK


def _map_vec(x):
    m = lax.rem(x, jnp.int32(11))
    m = jnp.where(m < 0, m + jnp.int32(11), m)
    return jnp.where(m == jnp.int32(10), jnp.int32(-1), m - jnp.int32(1))


@functools.partial(
    pl.kernel,
    mesh=plsc.VectorSubcoreMesh(core_axis_name="c", subcore_axis_name="s"),
    out_type=jax.ShapeDtypeStruct((_N,), jnp.int32),
    scratch_types=[
        pltpu.VMEM((2, _CHUNK), jnp.int32),
        pltpu.SemaphoreType.DMA,
        pltpu.SemaphoreType.DMA,
    ],
)
def _convert(sp_hbm, out_hbm, buf, sem_in, sem_out):
    wid = lax.axis_index("s") * _NC + lax.axis_index("c")
    base = wid * _PER_W

    def _start_in(g, slot):
        pltpu.async_copy(
            sp_hbm.at[pl.ds(base + g * _CHUNK, _CHUNK)], buf.at[slot], sem_in)

    def _compute(slot):
        def body(i, _):
            sl = pl.ds(pl.multiple_of(i * _L, _L), _L)
            buf[slot, sl] = _map_vec(buf[slot, sl])
            return 0
        lax.fori_loop(0, _CHUNK // _L, body, 0, unroll=4)

    def _start_out(g, slot):
        pltpu.async_copy(
            buf.at[slot], out_hbm.at[pl.ds(base + g * _CHUNK, _CHUNK)], sem_out)

    _start_in(0, 0)
    for g in range(_NCHUNK):
        slot = g % 2
        pltpu.make_async_copy(
            sp_hbm.at[pl.ds(0, _CHUNK)], buf.at[slot], sem_in).wait()
        if g + 1 < _NCHUNK:
            _start_in(g + 1, (g + 1) % 2)
        if g >= 2:
            pltpu.make_async_copy(
                buf.at[slot], out_hbm.at[pl.ds(0, _CHUNK)], sem_out).wait()
        _compute(slot)
        _start_out(g, slot)
    for g in (_NCHUNK - 2, _NCHUNK - 1):
        pltpu.make_async_copy(
            buf.at[g % 2], out_hbm.at[pl.ds(0, _CHUNK)], sem_out).wait()


def kernel(species, coordinates):
    converted = _convert(species.reshape(_N)).reshape(species.shape)
    return (converted, coordinates)
